# baseline (speedup 1.0000x reference)
import functools

import jax
import jax.numpy as jnp
from jax import lax
from jax.experimental import pallas as pl
from jax.experimental.pallas import tpu as pltpu
from jax.experimental.pallas import tpu_sc as plsc

N = 10000
NP = 10240
E = 160000
EP = 163840
NFEAT = 256
NHID = 256
NCLASS = 40
NC, NS, L = 2, 16, 16
CHUNK = 128
EDGES_PER_TILE = EP // NS
NCHUNK = EDGES_PER_TILE // CHUNK
ROWS_PER_TILE = NP // NS
DEG_ROWS = NP // (NC * NS)

_mesh = plsc.VectorSubcoreMesh(core_axis_name="c", subcore_axis_name="s")


NCHUNK_HALF = NCHUNK // 2


@functools.partial(
    pl.kernel,
    mesh=_mesh,
    out_type=jax.ShapeDtypeStruct((NC * NP,), jnp.float32),
    scratch_types=[
        pltpu.MemorySpace.VMEM_SHARED((NP, 128), jnp.float32),
        pltpu.VMEM((CHUNK, 128), jnp.float32),
        pltpu.VMEM((CHUNK, 128), jnp.float32),
        pltpu.VMEM((NCHUNK_HALF, CHUNK), jnp.int32),
        pltpu.VMEM((2 * L, 128), jnp.float32),
        pltpu.VMEM((ROWS_PER_TILE,), jnp.float32),
        pltpu.SemaphoreType.DMA,
        pltpu.SemaphoreType.DMA,
        pltpu.SemaphoreType.DMA,
    ],
)
def _deg_cnt(dst_hbm, cnt_hbm, cnt_sh, ones_v, zeros_v, didx_v, slab_v,
             dvec_v, sem, zsem, isem):
    c = lax.axis_index("c")
    s = lax.axis_index("s")

    row_base = pl.multiple_of((c * NS + s) * NCHUNK_HALF, 8)
    pltpu.async_copy(dst_hbm.at[pl.ds(row_base, NCHUNK_HALF)], didx_v, isem)

    @pl.loop(0, CHUNK)
    def _(i):
        for j in range(128 // L):
            zeros_v[i, pl.ds(j * L, L)] = jnp.zeros((L,), jnp.float32)

    r0 = s * ROWS_PER_TILE

    @pl.loop(0, ROWS_PER_TILE // CHUNK)
    def _(i):
        pltpu.async_copy(zeros_v, cnt_sh.at[pl.ds(r0 + i * CHUNK, CHUNK)], zsem)

    @pl.loop(0, CHUNK)
    def _(i):
        for j in range(128 // L):
            ones_v[i, pl.ds(j * L, L)] = jnp.ones((L,), jnp.float32)

    @pl.loop(0, ROWS_PER_TILE // CHUNK)
    def _(i):
        pltpu.make_async_copy(zeros_v, cnt_sh.at[pl.ds(r0 + i * CHUNK, CHUNK)],
                              zsem).wait()

    pltpu.make_async_copy(dst_hbm.at[pl.ds(row_base, NCHUNK_HALF)], didx_v,
                          isem).wait()

    plsc.subcore_barrier()

    @pl.loop(0, NCHUNK_HALF)
    def _(k):
        pltpu.async_copy(ones_v, cnt_sh.at[didx_v.at[k]], sem, add=True)

    @pl.loop(0, NCHUNK_HALF)
    def _(k):
        pltpu.make_async_copy(ones_v, cnt_sh.at[didx_v.at[k]], sem).wait()

    plsc.subcore_barrier()

    out0 = s * ROWS_PER_TILE
    lane = lax.iota(jnp.int32, L)

    @pl.loop(0, ROWS_PER_TILE // (2 * L))
    def _(t):
        pltpu.sync_copy(cnt_sh.at[pl.ds(out0 + t * (2 * L), 2 * L)], slab_v)
        for half in range(2):
            d = jnp.zeros((L,), jnp.float32)
            for l in range(L):
                d = jnp.where(lane == l, slab_v[half * L + l, pl.ds(0, L)], d)
            dvec_v[pl.ds(t * (2 * L) + half * L, L)] = d

    pltpu.sync_copy(dvec_v, cnt_hbm.at[pl.ds(c * NP + out0, ROWS_PER_TILE)])


CPS = 16
NSUP = NCHUNK // CPS


@functools.partial(
    pl.kernel,
    mesh=_mesh,
    out_type=jax.ShapeDtypeStruct((NC * NP, 128), jnp.float32),
    scratch_types=[
        pltpu.MemorySpace.VMEM_SHARED((NP, 128), jnp.float32),
        pltpu.VMEM((CHUNK, 128), jnp.float32),
        pltpu.VMEM((CHUNK, 128), jnp.float32),
        pltpu.VMEM((2 * CPS, CHUNK), jnp.int32),
        pltpu.VMEM((2 * CPS, CHUNK), jnp.int32),
        pltpu.SemaphoreType.DMA,
        pltpu.SemaphoreType.DMA,
        pltpu.SemaphoreType.DMA,
        pltpu.SemaphoreType.DMA,
        pltpu.SemaphoreType.DMA,
    ],
)
def _msg_pass(h_hbm, src_hbm, dst_hbm, out_hbm, acc_sh, rows_a, rows_b,
              sidx_v, didx_v, gsa, gsb, ssa, ssb, ists):
    c = lax.axis_index("c")
    s = lax.axis_index("s")

    @pl.loop(0, CHUNK)
    def _(i):
        for j in range(128 // L):
            rows_a[i, pl.ds(j * L, L)] = jnp.zeros((L,), jnp.float32)

    r0 = s * ROWS_PER_TILE

    @pl.loop(0, ROWS_PER_TILE // CHUNK)
    def _(i):
        pltpu.sync_copy(rows_a, acc_sh.at[pl.ds(r0 + i * CHUNK, CHUNK)])

    plsc.subcore_barrier()

    row_base = s * (EDGES_PER_TILE // CHUNK)
    hoff = c * NP

    def stage_refs(sup):
        slot = pl.multiple_of((sup % 2) * CPS, CPS)
        off = pl.multiple_of(row_base + sup * CPS, CPS)
        return (src_hbm.at[pl.ds(off, CPS)], sidx_v.at[pl.ds(slot, CPS)],
                dst_hbm.at[pl.ds(off, CPS)], didx_v.at[pl.ds(slot, CPS)])

    def stage_start(sup):
        s_src, d_src, s_dst, d_dst = stage_refs(sup)
        pltpu.async_copy(s_src, d_src, ists)
        pltpu.async_copy(s_dst, d_dst, ists)

    def stage_finish(sup):
        s_src, d_src, s_dst, d_dst = stage_refs(sup)
        pltpu.make_async_copy(s_src, d_src, ists).wait()
        pltpu.make_async_copy(s_dst, d_dst, ists).wait()
        slot = (sup % 2) * CPS

        @pl.loop(0, CPS)
        def _(j):
            for jj in range(CHUNK // L):
                sidx_v[slot + j, pl.ds(jj * L, L)] = (
                    sidx_v[slot + j, pl.ds(jj * L, L)] + hoff)

    def idx_row(k):
        return ((k // CPS) % 2) * CPS + (k % CPS)

    def gather(k, buf, sem):
        pltpu.async_copy(h_hbm.at[sidx_v.at[idx_row(k)]], buf, sem)

    def scatter(k, buf, sem):
        pltpu.async_copy(buf, acc_sh.at[didx_v.at[idx_row(k)]], sem, add=True)

    stage_start(0)
    stage_finish(0)
    gather(0, rows_a, gsa)

    @pl.loop(0, NCHUNK // 2)
    def _(t):
        k = t * 2
        pltpu.make_async_copy(h_hbm.at[sidx_v.at[idx_row(k)]], rows_a, gsa).wait()

        @pl.when(t > 0)
        def _():
            pltpu.make_async_copy(rows_b, acc_sh.at[didx_v.at[idx_row(k)]],
                                  ssb).wait()

        @pl.when((k % CPS == 0) & (k < NCHUNK - CPS))
        def _():
            stage_start(k // CPS + 1)

        gather(k + 1, rows_b, gsb)
        scatter(k, rows_a, ssa)

        pltpu.make_async_copy(h_hbm.at[sidx_v.at[idx_row(k + 1)]], rows_b,
                              gsb).wait()
        scatter(k + 1, rows_b, ssb)

        pltpu.make_async_copy(rows_a, acc_sh.at[didx_v.at[idx_row(k)]],
                              ssa).wait()

        @pl.when((k % CPS == CPS - 2) & (k + 2 < NCHUNK))
        def _():
            stage_finish((k + 2) // CPS)

        @pl.when(k + 2 < NCHUNK)
        def _():
            gather(k + 2, rows_a, gsa)

    pltpu.make_async_copy(rows_b, acc_sh.at[didx_v.at[idx_row(NCHUNK - 1)]],
                          ssb).wait()

    plsc.subcore_barrier()
    pltpu.sync_copy(acc_sh.at[pl.ds(r0, ROWS_PER_TILE)],
                    out_hbm.at[pl.ds(c * NP + r0, ROWS_PER_TILE)])


BR = 2048


def _mm1_body(x_ref, w_ref, ca_ref, cb_ref, out_ref, dinv_ref):
    h = jnp.dot(x_ref[...], w_ref[...], preferred_element_type=jnp.float32)
    d = lax.rsqrt(ca_ref[0] + cb_ref[0] + 1.0)
    dinv_ref[...] = d
    out_ref[0] = h[:, :128] * d
    out_ref[1] = h[:, 128:] * d


def _mm1(xp, W1, cA, cB):
    return pl.pallas_call(
        _mm1_body,
        grid=(NP // BR,),
        in_specs=[
            pl.BlockSpec((BR, NFEAT), lambda i: (i, 0)),
            pl.BlockSpec((NFEAT, NHID), lambda i: (0, 0)),
            pl.BlockSpec((1, BR, 1), lambda i: (0, i, 0)),
            pl.BlockSpec((1, BR, 1), lambda i: (1, i, 0)),
        ],
        out_specs=[
            pl.BlockSpec((2, BR, 128), lambda i: (0, i, 0)),
            pl.BlockSpec((BR, 1), lambda i: (i, 0)),
        ],
        out_shape=[
            jax.ShapeDtypeStruct((2, NP, 128), jnp.float32),
            jax.ShapeDtypeStruct((NP, 1), jnp.float32),
        ],
    )(xp, W1, cA, cB)


def _mm2_body(acc_ref, g_ref, dinv_ref, b_ref, w_ref, out_ref):
    dinv = dinv_ref[...]
    a0 = jnp.maximum((acc_ref[0] + g_ref[0]) * dinv + b_ref[0, 0], 0.0)
    a1 = jnp.maximum((acc_ref[1] + g_ref[1]) * dinv + b_ref[1, 0], 0.0)
    p = (jnp.dot(a0, w_ref[0], preferred_element_type=jnp.float32)
         + jnp.dot(a1, w_ref[1], preferred_element_type=jnp.float32))
    out_ref[0] = p * dinv


def _mm2(acc1, g1, dinv2d, b1r, W2r):
    return pl.pallas_call(
        _mm2_body,
        grid=(NP // BR, 2),
        in_specs=[
            pl.BlockSpec((2, BR, 128), lambda i, j: (0, i, 0)),
            pl.BlockSpec((2, BR, 128), lambda i, j: (0, i, 0)),
            pl.BlockSpec((BR, 1), lambda i, j: (i, 0)),
            pl.BlockSpec((2, 1, 128), lambda i, j: (0, 0, 0)),
            pl.BlockSpec((2, 128, 128), lambda i, j: (0, 0, j)),
        ],
        out_specs=pl.BlockSpec((1, BR, 128), lambda i, j: (j, i, 0)),
        out_shape=jax.ShapeDtypeStruct((2, NP, 128), jnp.float32),
    )(acc1, g1, dinv2d, b1r, W2r)


def _mm3_body(acc_ref, g_ref, dinv_ref, b_ref, w_ref, bfc_ref, out_ref):
    dinv = dinv_ref[...]
    a0 = (acc_ref[0] + g_ref[0]) * dinv + b_ref[0, 0]
    a1 = (acc_ref[1] + g_ref[1]) * dinv + b_ref[1, 0]
    out_ref[...] = (jnp.dot(a0, w_ref[0], preferred_element_type=jnp.float32)
                    + jnp.dot(a1, w_ref[1], preferred_element_type=jnp.float32)
                    + jnp.broadcast_to(bfc_ref[...], out_ref.shape))


def _mm3(acc2, g2, dinv2d, b2r, Wfcr, bfc2d):
    return pl.pallas_call(
        _mm3_body,
        grid=(NP // BR,),
        in_specs=[
            pl.BlockSpec((2, BR, 128), lambda i: (0, i, 0)),
            pl.BlockSpec((2, BR, 128), lambda i: (0, i, 0)),
            pl.BlockSpec((BR, 1), lambda i: (i, 0)),
            pl.BlockSpec((2, 1, 128), lambda i: (0, 0, 0)),
            pl.BlockSpec((2, 128, NCLASS), lambda i: (0, 0, 0)),
            pl.BlockSpec((1, NCLASS), lambda i: (0, 0)),
        ],
        out_specs=pl.BlockSpec((BR, NCLASS), lambda i: (i, 0)),
        out_shape=jax.ShapeDtypeStruct((NP, NCLASS), jnp.float32),
    )(acc2, g2, dinv2d, b2r, Wfcr, bfc2d)


def kernel(x, edge_index, W1, b1, W2, b2, Wfc, bfc):
    x = x.astype(jnp.float32)
    src = edge_index[0].astype(jnp.int32)
    dst = edge_index[1].astype(jnp.int32)
    srcp = jnp.concatenate([src, jnp.zeros((EP - E,), jnp.int32)])
    dstp = jnp.concatenate([dst, jnp.full((EP - E,), N + 16, jnp.int32)])
    srcp = srcp.reshape(EP // CHUNK, CHUNK)
    dstp = dstp.reshape(EP // CHUNK, CHUNK)
    xp = jnp.zeros((NP, NFEAT), jnp.float32).at[:N].set(x)

    cnts = _deg_cnt(dstp).reshape(NC, NP, 1)

    g1, dinv2d = _mm1(xp, W1, cnts, cnts)
    acc1 = _msg_pass(g1.reshape(NC * NP, 128), srcp, dstp)
    g2 = _mm2(acc1.reshape(2, NP, 128), g1, dinv2d,
              b1.reshape(2, 1, 128), W2.reshape(2, 128, NHID))
    acc2 = _msg_pass(g2.reshape(NC * NP, 128), srcp, dstp)
    out = _mm3(acc2.reshape(2, NP, 128), g2, dinv2d,
               b2.reshape(2, 1, 128), Wfc.reshape(2, 128, NCLASS),
               bfc.reshape(1, NCLASS))
    return out[:N]


# speedup vs baseline: 1.1656x; 1.1656x over previous
import functools

import jax
import jax.numpy as jnp
from jax import lax
from jax.experimental import pallas as pl
from jax.experimental.pallas import tpu as pltpu
from jax.experimental.pallas import tpu_sc as plsc

N = 10000
NP = 10240
E = 160000
EP = 163840
NFEAT = 256
NHID = 256
NCLASS = 40
NC, NS, L = 2, 16, 16
CHUNK = 128
EDGES_PER_TILE = EP // NS
NCHUNK = EDGES_PER_TILE // CHUNK
ROWS_PER_TILE = NP // NS
DEG_ROWS = NP // (NC * NS)

_mesh = plsc.VectorSubcoreMesh(core_axis_name="c", subcore_axis_name="s")


NCHUNK_HALF = NCHUNK // 2


@functools.partial(
    pl.kernel,
    mesh=_mesh,
    out_type=jax.ShapeDtypeStruct((NC * NP,), jnp.float32),
    scratch_types=[
        pltpu.MemorySpace.VMEM_SHARED((NP, 128), jnp.float32),
        pltpu.VMEM((CHUNK, 128), jnp.float32),
        pltpu.VMEM((CHUNK, 128), jnp.float32),
        pltpu.VMEM((NCHUNK_HALF, CHUNK), jnp.int32),
        pltpu.VMEM((2 * L, 128), jnp.float32),
        pltpu.VMEM((ROWS_PER_TILE,), jnp.float32),
        pltpu.SemaphoreType.DMA,
        pltpu.SemaphoreType.DMA,
        pltpu.SemaphoreType.DMA,
    ],
)
def _deg_cnt(dst_hbm, cnt_hbm, cnt_sh, ones_v, zeros_v, didx_v, slab_v,
             dvec_v, sem, zsem, isem):
    c = lax.axis_index("c")
    s = lax.axis_index("s")

    row_base = pl.multiple_of((c * NS + s) * NCHUNK_HALF, 8)
    pltpu.async_copy(dst_hbm.at[pl.ds(row_base, NCHUNK_HALF)], didx_v, isem)

    @pl.loop(0, CHUNK)
    def _(i):
        for j in range(128 // L):
            zeros_v[i, pl.ds(j * L, L)] = jnp.zeros((L,), jnp.float32)

    r0 = s * ROWS_PER_TILE

    @pl.loop(0, ROWS_PER_TILE // CHUNK)
    def _(i):
        pltpu.async_copy(zeros_v, cnt_sh.at[pl.ds(r0 + i * CHUNK, CHUNK)], zsem)

    @pl.loop(0, CHUNK)
    def _(i):
        for j in range(128 // L):
            ones_v[i, pl.ds(j * L, L)] = jnp.ones((L,), jnp.float32)

    @pl.loop(0, ROWS_PER_TILE // CHUNK)
    def _(i):
        pltpu.make_async_copy(zeros_v, cnt_sh.at[pl.ds(r0 + i * CHUNK, CHUNK)],
                              zsem).wait()

    pltpu.make_async_copy(dst_hbm.at[pl.ds(row_base, NCHUNK_HALF)], didx_v,
                          isem).wait()

    plsc.subcore_barrier()

    @pl.loop(0, NCHUNK_HALF)
    def _(k):
        pltpu.async_copy(ones_v, cnt_sh.at[didx_v.at[k]], sem, add=True)

    @pl.loop(0, NCHUNK_HALF)
    def _(k):
        pltpu.make_async_copy(ones_v, cnt_sh.at[didx_v.at[k]], sem).wait()

    plsc.subcore_barrier()

    out0 = s * ROWS_PER_TILE
    lane = lax.iota(jnp.int32, L)

    @pl.loop(0, ROWS_PER_TILE // (2 * L))
    def _(t):
        pltpu.sync_copy(cnt_sh.at[pl.ds(out0 + t * (2 * L), 2 * L)], slab_v)
        for half in range(2):
            d = jnp.zeros((L,), jnp.float32)
            for l in range(L):
                d = jnp.where(lane == l, slab_v[half * L + l, pl.ds(0, L)], d)
            dvec_v[pl.ds(t * (2 * L) + half * L, L)] = d

    pltpu.sync_copy(dvec_v, cnt_hbm.at[pl.ds(c * NP + out0, ROWS_PER_TILE)])


CPS = 16
NSUP = NCHUNK // CPS


@functools.partial(
    pl.kernel,
    mesh=_mesh,
    out_type=jax.ShapeDtypeStruct((NC * NP, 128), jnp.float32),
    scratch_types=[
        pltpu.MemorySpace.VMEM_SHARED((NP, 128), jnp.float32),
        pltpu.VMEM((CHUNK, 128), jnp.float32),
        pltpu.VMEM((CHUNK, 128), jnp.float32),
        pltpu.VMEM((2 * CPS, CHUNK), jnp.int32),
        pltpu.VMEM((2 * CPS, CHUNK), jnp.int32),
        pltpu.SemaphoreType.DMA,
        pltpu.SemaphoreType.DMA,
        pltpu.SemaphoreType.DMA,
        pltpu.SemaphoreType.DMA,
        pltpu.SemaphoreType.DMA,
    ],
)
def _msg_pass(h_hbm, src_hbm, dst_hbm, out_hbm, acc_sh, rows_a, rows_b,
              sidx_v, didx_v, gsa, gsb, ssa, ssb, ists):
    c = lax.axis_index("c")
    s = lax.axis_index("s")

    @pl.loop(0, CHUNK)
    def _(i):
        for j in range(128 // L):
            rows_a[i, pl.ds(j * L, L)] = jnp.zeros((L,), jnp.float32)

    r0 = s * ROWS_PER_TILE

    @pl.loop(0, ROWS_PER_TILE // CHUNK)
    def _(i):
        pltpu.sync_copy(rows_a, acc_sh.at[pl.ds(r0 + i * CHUNK, CHUNK)])

    plsc.subcore_barrier()

    row_base = s * (EDGES_PER_TILE // CHUNK)
    hoff = c * NP

    def stage_refs(sup):
        slot = pl.multiple_of((sup % 2) * CPS, CPS)
        off = pl.multiple_of(row_base + sup * CPS, CPS)
        return (src_hbm.at[pl.ds(off, CPS)], sidx_v.at[pl.ds(slot, CPS)],
                dst_hbm.at[pl.ds(off, CPS)], didx_v.at[pl.ds(slot, CPS)])

    def stage_start(sup):
        s_src, d_src, s_dst, d_dst = stage_refs(sup)
        pltpu.async_copy(s_src, d_src, ists)
        pltpu.async_copy(s_dst, d_dst, ists)

    def stage_finish(sup):
        s_src, d_src, s_dst, d_dst = stage_refs(sup)
        pltpu.make_async_copy(s_src, d_src, ists).wait()
        pltpu.make_async_copy(s_dst, d_dst, ists).wait()
        slot = (sup % 2) * CPS

        @pl.loop(0, CPS)
        def _(j):
            for jj in range(CHUNK // L):
                sidx_v[slot + j, pl.ds(jj * L, L)] = (
                    sidx_v[slot + j, pl.ds(jj * L, L)] + hoff)

    def idx_row(k):
        return ((k // CPS) % 2) * CPS + (k % CPS)

    def gather(k, buf, sem):
        pltpu.async_copy(h_hbm.at[sidx_v.at[idx_row(k)]], buf, sem)

    def scatter(k, buf, sem):
        pltpu.async_copy(buf, acc_sh.at[didx_v.at[idx_row(k)]], sem, add=True)

    stage_start(0)
    stage_finish(0)
    gather(0, rows_a, gsa)

    @pl.loop(0, NCHUNK // 2)
    def _(t):
        k = t * 2
        pltpu.make_async_copy(h_hbm.at[sidx_v.at[idx_row(k)]], rows_a, gsa).wait()

        @pl.when(t > 0)
        def _():
            pltpu.make_async_copy(rows_b, acc_sh.at[didx_v.at[idx_row(k)]],
                                  ssb).wait()

        @pl.when((k % CPS == 0) & (k < NCHUNK - CPS))
        def _():
            stage_start(k // CPS + 1)

        gather(k + 1, rows_b, gsb)
        scatter(k, rows_a, ssa)

        pltpu.make_async_copy(h_hbm.at[sidx_v.at[idx_row(k + 1)]], rows_b,
                              gsb).wait()
        scatter(k + 1, rows_b, ssb)

        pltpu.make_async_copy(rows_a, acc_sh.at[didx_v.at[idx_row(k)]],
                              ssa).wait()

        @pl.when((k % CPS == CPS - 2) & (k + 2 < NCHUNK))
        def _():
            stage_finish((k + 2) // CPS)

        @pl.when(k + 2 < NCHUNK)
        def _():
            gather(k + 2, rows_a, gsa)

    pltpu.make_async_copy(rows_b, acc_sh.at[didx_v.at[idx_row(NCHUNK - 1)]],
                          ssb).wait()

    plsc.subcore_barrier()
    pltpu.sync_copy(acc_sh.at[pl.ds(r0, ROWS_PER_TILE)],
                    out_hbm.at[pl.ds(c * NP + r0, ROWS_PER_TILE)])


BR = 2048


def _mm1_body(x_ref, w_ref, ca_ref, cb_ref, out_ref, dinv_ref):
    h = jnp.dot(x_ref[...], w_ref[...], preferred_element_type=jnp.float32)
    d = lax.rsqrt(ca_ref[...] + cb_ref[...] + 1.0)
    dinv_ref[...] = d
    out_ref[0] = h[:, :128] * d
    out_ref[1] = h[:, 128:] * d


def _mm1(xp, W1, cA, cB):
    return pl.pallas_call(
        _mm1_body,
        grid=(NP // BR,),
        in_specs=[
            pl.BlockSpec((BR, NFEAT), lambda i: (i, 0)),
            pl.BlockSpec((NFEAT, NHID), lambda i: (0, 0)),
            pl.BlockSpec((BR, 1), lambda i: (i, 0)),
            pl.BlockSpec((BR, 1), lambda i: (i, 0)),
        ],
        out_specs=[
            pl.BlockSpec((2, BR, 128), lambda i: (0, i, 0)),
            pl.BlockSpec((BR, 1), lambda i: (i, 0)),
        ],
        out_shape=[
            jax.ShapeDtypeStruct((2, NP, 128), jnp.float32),
            jax.ShapeDtypeStruct((NP, 1), jnp.float32),
        ],
    )(xp, W1, cA, cB)


def _mm2_body(acc_ref, g_ref, dinv_ref, b_ref, w_ref, out_ref):
    dinv = dinv_ref[...]
    a0 = jnp.maximum((acc_ref[0] + g_ref[0]) * dinv + b_ref[0, 0], 0.0)
    a1 = jnp.maximum((acc_ref[1] + g_ref[1]) * dinv + b_ref[1, 0], 0.0)
    p = (jnp.dot(a0, w_ref[0], preferred_element_type=jnp.float32)
         + jnp.dot(a1, w_ref[1], preferred_element_type=jnp.float32))
    out_ref[0] = p * dinv


def _mm2(acc1, g1, dinv2d, b1r, W2r):
    return pl.pallas_call(
        _mm2_body,
        grid=(NP // BR, 2),
        in_specs=[
            pl.BlockSpec((2, BR, 128), lambda i, j: (0, i, 0)),
            pl.BlockSpec((2, BR, 128), lambda i, j: (0, i, 0)),
            pl.BlockSpec((BR, 1), lambda i, j: (i, 0)),
            pl.BlockSpec((2, 1, 128), lambda i, j: (0, 0, 0)),
            pl.BlockSpec((2, 128, 128), lambda i, j: (0, 0, j)),
        ],
        out_specs=pl.BlockSpec((1, BR, 128), lambda i, j: (j, i, 0)),
        out_shape=jax.ShapeDtypeStruct((2, NP, 128), jnp.float32),
    )(acc1, g1, dinv2d, b1r, W2r)


def _mm3_body(acc_ref, g_ref, dinv_ref, b_ref, w_ref, bfc_ref, out_ref):
    dinv = dinv_ref[...]
    a0 = (acc_ref[0] + g_ref[0]) * dinv + b_ref[0, 0]
    a1 = (acc_ref[1] + g_ref[1]) * dinv + b_ref[1, 0]
    out_ref[...] = (jnp.dot(a0, w_ref[0], preferred_element_type=jnp.float32)
                    + jnp.dot(a1, w_ref[1], preferred_element_type=jnp.float32)
                    + jnp.broadcast_to(bfc_ref[...], out_ref.shape))


def _mm3(acc2, g2, dinv2d, b2r, Wfcr, bfc2d):
    return pl.pallas_call(
        _mm3_body,
        grid=(NP // BR,),
        in_specs=[
            pl.BlockSpec((2, BR, 128), lambda i: (0, i, 0)),
            pl.BlockSpec((2, BR, 128), lambda i: (0, i, 0)),
            pl.BlockSpec((BR, 1), lambda i: (i, 0)),
            pl.BlockSpec((2, 1, 128), lambda i: (0, 0, 0)),
            pl.BlockSpec((2, 128, NCLASS), lambda i: (0, 0, 0)),
            pl.BlockSpec((1, NCLASS), lambda i: (0, 0)),
        ],
        out_specs=pl.BlockSpec((BR, NCLASS), lambda i: (i, 0)),
        out_shape=jax.ShapeDtypeStruct((NP, NCLASS), jnp.float32),
    )(acc2, g2, dinv2d, b2r, Wfcr, bfc2d)


def kernel(x, edge_index, W1, b1, W2, b2, Wfc, bfc):
    x = x.astype(jnp.float32)
    src = edge_index[0].astype(jnp.int32)
    dst = edge_index[1].astype(jnp.int32)
    srcp = jnp.concatenate([src, jnp.zeros((EP - E,), jnp.int32)])
    dstp = jnp.concatenate([dst, jnp.full((EP - E,), N + 16, jnp.int32)])
    srcp = srcp.reshape(EP // CHUNK, CHUNK)
    dstp = dstp.reshape(EP // CHUNK, CHUNK)
    xp = jnp.zeros((NP, NFEAT), jnp.float32).at[:N].set(x)

    cnts = _deg_cnt(dstp).reshape(NC, NP, 1)

    g1, dinv2d = _mm1(xp, W1, cnts[0], cnts[1])
    acc1 = _msg_pass(g1.reshape(NC * NP, 128), srcp, dstp)
    g2 = _mm2(acc1.reshape(2, NP, 128), g1, dinv2d,
              b1.reshape(2, 1, 128), W2.reshape(2, 128, NHID))
    acc2 = _msg_pass(g2.reshape(NC * NP, 128), srcp, dstp)
    out = _mm3(acc2.reshape(2, NP, 128), g2, dinv2d,
               b2.reshape(2, 1, 128), Wfc.reshape(2, 128, NCLASS),
               bfc.reshape(1, NCLASS))
    return out[:N]


# speedup vs baseline: 1.1989x; 1.0286x over previous
import functools

import jax
import jax.numpy as jnp
from jax import lax
from jax.experimental import pallas as pl
from jax.experimental.pallas import tpu as pltpu
from jax.experimental.pallas import tpu_sc as plsc

N = 10000
NP = 10240
E = 160000
EP = 163840
NFEAT = 256
NHID = 256
NCLASS = 40
NC, NS, L = 2, 16, 16
CHUNK = 128
EDGES_PER_TILE = EP // NS
NCHUNK = EDGES_PER_TILE // CHUNK
ROWS_PER_TILE = NP // NS
DEG_ROWS = NP // (NC * NS)

_mesh = plsc.VectorSubcoreMesh(core_axis_name="c", subcore_axis_name="s")


NCHUNK_HALF = NCHUNK // 2


@functools.partial(
    pl.kernel,
    mesh=_mesh,
    out_type=[jax.ShapeDtypeStruct((NP,), jnp.float32),
              jax.ShapeDtypeStruct((NP,), jnp.float32)],
    scratch_types=[
        pltpu.MemorySpace.VMEM_SHARED((NP, 128), jnp.float32),
        pltpu.VMEM((CHUNK, 128), jnp.float32),
        pltpu.VMEM((CHUNK, 128), jnp.float32),
        pltpu.VMEM((NCHUNK_HALF, CHUNK), jnp.int32),
        pltpu.VMEM((2 * L, 128), jnp.float32),
        pltpu.VMEM((ROWS_PER_TILE,), jnp.float32),
        pltpu.SemaphoreType.DMA,
        pltpu.SemaphoreType.DMA,
        pltpu.SemaphoreType.DMA,
    ],
)
def _deg_cnt(dst_hbm, cnta_hbm, cntb_hbm, cnt_sh, ones_v, zeros_v, didx_v,
             slab_v, dvec_v, sem, zsem, isem):
    c = lax.axis_index("c")
    s = lax.axis_index("s")

    row_base = pl.multiple_of((c * NS + s) * NCHUNK_HALF, 8)
    pltpu.async_copy(dst_hbm.at[pl.ds(row_base, NCHUNK_HALF)], didx_v, isem)

    @pl.loop(0, CHUNK)
    def _(i):
        for j in range(128 // L):
            zeros_v[i, pl.ds(j * L, L)] = jnp.zeros((L,), jnp.float32)

    r0 = s * ROWS_PER_TILE

    @pl.loop(0, ROWS_PER_TILE // CHUNK)
    def _(i):
        pltpu.async_copy(zeros_v, cnt_sh.at[pl.ds(r0 + i * CHUNK, CHUNK)], zsem)

    @pl.loop(0, CHUNK)
    def _(i):
        for j in range(128 // L):
            ones_v[i, pl.ds(j * L, L)] = jnp.ones((L,), jnp.float32)

    @pl.loop(0, ROWS_PER_TILE // CHUNK)
    def _(i):
        pltpu.make_async_copy(zeros_v, cnt_sh.at[pl.ds(r0 + i * CHUNK, CHUNK)],
                              zsem).wait()

    pltpu.make_async_copy(dst_hbm.at[pl.ds(row_base, NCHUNK_HALF)], didx_v,
                          isem).wait()

    plsc.subcore_barrier()

    @pl.loop(0, NCHUNK_HALF)
    def _(k):
        pltpu.async_copy(ones_v, cnt_sh.at[didx_v.at[k]], sem, add=True)

    @pl.loop(0, NCHUNK_HALF)
    def _(k):
        pltpu.make_async_copy(ones_v, cnt_sh.at[didx_v.at[k]], sem).wait()

    plsc.subcore_barrier()

    out0 = s * ROWS_PER_TILE
    lane = lax.iota(jnp.int32, L)

    @pl.loop(0, ROWS_PER_TILE // (2 * L))
    def _(t):
        pltpu.sync_copy(cnt_sh.at[pl.ds(out0 + t * (2 * L), 2 * L)], slab_v)
        for half in range(2):
            d = jnp.zeros((L,), jnp.float32)
            for l in range(L):
                d = jnp.where(lane == l, slab_v[half * L + l, pl.ds(0, L)], d)
            dvec_v[pl.ds(t * (2 * L) + half * L, L)] = d

    @pl.when(c == 0)
    def _():
        pltpu.sync_copy(dvec_v, cnta_hbm.at[pl.ds(out0, ROWS_PER_TILE)])

    @pl.when(c == 1)
    def _():
        pltpu.sync_copy(dvec_v, cntb_hbm.at[pl.ds(out0, ROWS_PER_TILE)])


CPS = 16
NSUP = NCHUNK // CPS


@functools.partial(
    pl.kernel,
    mesh=_mesh,
    out_type=jax.ShapeDtypeStruct((NC * NP, 128), jnp.float32),
    scratch_types=[
        pltpu.MemorySpace.VMEM_SHARED((NP, 128), jnp.float32),
        pltpu.VMEM((CHUNK, 128), jnp.float32),
        pltpu.VMEM((CHUNK, 128), jnp.float32),
        pltpu.VMEM((2 * CPS, CHUNK), jnp.int32),
        pltpu.VMEM((2 * CPS, CHUNK), jnp.int32),
        pltpu.SemaphoreType.DMA,
        pltpu.SemaphoreType.DMA,
        pltpu.SemaphoreType.DMA,
        pltpu.SemaphoreType.DMA,
        pltpu.SemaphoreType.DMA,
    ],
)
def _msg_pass(h_hbm, src_hbm, dst_hbm, out_hbm, acc_sh, rows_a, rows_b,
              sidx_v, didx_v, gsa, gsb, ssa, ssb, ists):
    c = lax.axis_index("c")
    s = lax.axis_index("s")

    @pl.loop(0, CHUNK)
    def _(i):
        for j in range(128 // L):
            rows_a[i, pl.ds(j * L, L)] = jnp.zeros((L,), jnp.float32)

    r0 = s * ROWS_PER_TILE

    @pl.loop(0, ROWS_PER_TILE // CHUNK)
    def _(i):
        pltpu.sync_copy(rows_a, acc_sh.at[pl.ds(r0 + i * CHUNK, CHUNK)])

    plsc.subcore_barrier()

    row_base = s * (EDGES_PER_TILE // CHUNK)
    hoff = c * NP

    def stage_refs(sup):
        slot = pl.multiple_of((sup % 2) * CPS, CPS)
        off = pl.multiple_of(row_base + sup * CPS, CPS)
        return (src_hbm.at[pl.ds(off, CPS)], sidx_v.at[pl.ds(slot, CPS)],
                dst_hbm.at[pl.ds(off, CPS)], didx_v.at[pl.ds(slot, CPS)])

    def stage_start(sup):
        s_src, d_src, s_dst, d_dst = stage_refs(sup)
        pltpu.async_copy(s_src, d_src, ists)
        pltpu.async_copy(s_dst, d_dst, ists)

    def stage_finish(sup):
        s_src, d_src, s_dst, d_dst = stage_refs(sup)
        pltpu.make_async_copy(s_src, d_src, ists).wait()
        pltpu.make_async_copy(s_dst, d_dst, ists).wait()
        slot = (sup % 2) * CPS

        @pl.loop(0, CPS)
        def _(j):
            for jj in range(CHUNK // L):
                sidx_v[slot + j, pl.ds(jj * L, L)] = (
                    sidx_v[slot + j, pl.ds(jj * L, L)] + hoff)

    def idx_row(k):
        return ((k // CPS) % 2) * CPS + (k % CPS)

    def gather(k, buf, sem):
        pltpu.async_copy(h_hbm.at[sidx_v.at[idx_row(k)]], buf, sem)

    def scatter(k, buf, sem):
        pltpu.async_copy(buf, acc_sh.at[didx_v.at[idx_row(k)]], sem, add=True)

    stage_start(0)
    stage_finish(0)
    gather(0, rows_a, gsa)

    @pl.loop(0, NCHUNK // 2)
    def _(t):
        k = t * 2
        pltpu.make_async_copy(h_hbm.at[sidx_v.at[idx_row(k)]], rows_a, gsa).wait()

        @pl.when(t > 0)
        def _():
            pltpu.make_async_copy(rows_b, acc_sh.at[didx_v.at[idx_row(k)]],
                                  ssb).wait()

        @pl.when((k % CPS == 0) & (k < NCHUNK - CPS))
        def _():
            stage_start(k // CPS + 1)

        gather(k + 1, rows_b, gsb)
        scatter(k, rows_a, ssa)

        pltpu.make_async_copy(h_hbm.at[sidx_v.at[idx_row(k + 1)]], rows_b,
                              gsb).wait()
        scatter(k + 1, rows_b, ssb)

        pltpu.make_async_copy(rows_a, acc_sh.at[didx_v.at[idx_row(k)]],
                              ssa).wait()

        @pl.when((k % CPS == CPS - 2) & (k + 2 < NCHUNK))
        def _():
            stage_finish((k + 2) // CPS)

        @pl.when(k + 2 < NCHUNK)
        def _():
            gather(k + 2, rows_a, gsa)

    pltpu.make_async_copy(rows_b, acc_sh.at[didx_v.at[idx_row(NCHUNK - 1)]],
                          ssb).wait()

    plsc.subcore_barrier()
    pltpu.sync_copy(acc_sh.at[pl.ds(r0, ROWS_PER_TILE)],
                    out_hbm.at[pl.ds(c * NP + r0, ROWS_PER_TILE)])


BR = 2048


def _mm1_body(x_ref, w_ref, ca_ref, cb_ref, out_ref, dinv_ref):
    h = jnp.dot(x_ref[...], w_ref[...], preferred_element_type=jnp.float32)
    d = lax.rsqrt(ca_ref[...] + cb_ref[...] + 1.0)
    dinv_ref[...] = d
    out_ref[0] = h[:, :128] * d
    out_ref[1] = h[:, 128:] * d


def _mm1(xp, W1, cA, cB):
    return pl.pallas_call(
        _mm1_body,
        grid=(NP // BR,),
        in_specs=[
            pl.BlockSpec((BR, NFEAT), lambda i: (i, 0)),
            pl.BlockSpec((NFEAT, NHID), lambda i: (0, 0)),
            pl.BlockSpec((BR, 1), lambda i: (i, 0)),
            pl.BlockSpec((BR, 1), lambda i: (i, 0)),
        ],
        out_specs=[
            pl.BlockSpec((2, BR, 128), lambda i: (0, i, 0)),
            pl.BlockSpec((BR, 1), lambda i: (i, 0)),
        ],
        out_shape=[
            jax.ShapeDtypeStruct((2, NP, 128), jnp.float32),
            jax.ShapeDtypeStruct((NP, 1), jnp.float32),
        ],
    )(xp, W1, cA, cB)


def _mm2_body(acc_ref, g_ref, dinv_ref, b_ref, w_ref, out_ref):
    dinv = dinv_ref[...]
    a0 = jnp.maximum((acc_ref[0] + g_ref[0]) * dinv + b_ref[0, 0], 0.0)
    a1 = jnp.maximum((acc_ref[1] + g_ref[1]) * dinv + b_ref[1, 0], 0.0)
    p = (jnp.dot(a0, w_ref[0], preferred_element_type=jnp.float32)
         + jnp.dot(a1, w_ref[1], preferred_element_type=jnp.float32))
    out_ref[0] = p * dinv


def _mm2(acc1, g1, dinv2d, b1r, W2r):
    return pl.pallas_call(
        _mm2_body,
        grid=(NP // BR, 2),
        in_specs=[
            pl.BlockSpec((2, BR, 128), lambda i, j: (0, i, 0)),
            pl.BlockSpec((2, BR, 128), lambda i, j: (0, i, 0)),
            pl.BlockSpec((BR, 1), lambda i, j: (i, 0)),
            pl.BlockSpec((2, 1, 128), lambda i, j: (0, 0, 0)),
            pl.BlockSpec((2, 128, 128), lambda i, j: (0, 0, j)),
        ],
        out_specs=pl.BlockSpec((1, BR, 128), lambda i, j: (j, i, 0)),
        out_shape=jax.ShapeDtypeStruct((2, NP, 128), jnp.float32),
    )(acc1, g1, dinv2d, b1r, W2r)


def _mm3_body(acc_ref, g_ref, dinv_ref, b_ref, w_ref, bfc_ref, out_ref):
    dinv = dinv_ref[...]
    a0 = (acc_ref[0] + g_ref[0]) * dinv + b_ref[0, 0]
    a1 = (acc_ref[1] + g_ref[1]) * dinv + b_ref[1, 0]
    out_ref[...] = (jnp.dot(a0, w_ref[0], preferred_element_type=jnp.float32)
                    + jnp.dot(a1, w_ref[1], preferred_element_type=jnp.float32)
                    + jnp.broadcast_to(bfc_ref[...], out_ref.shape))


def _mm3(acc2, g2, dinv2d, b2r, Wfcr, bfc2d):
    return pl.pallas_call(
        _mm3_body,
        grid=(NP // BR,),
        in_specs=[
            pl.BlockSpec((2, BR, 128), lambda i: (0, i, 0)),
            pl.BlockSpec((2, BR, 128), lambda i: (0, i, 0)),
            pl.BlockSpec((BR, 1), lambda i: (i, 0)),
            pl.BlockSpec((2, 1, 128), lambda i: (0, 0, 0)),
            pl.BlockSpec((2, 128, NCLASS), lambda i: (0, 0, 0)),
            pl.BlockSpec((1, NCLASS), lambda i: (0, 0)),
        ],
        out_specs=pl.BlockSpec((BR, NCLASS), lambda i: (i, 0)),
        out_shape=jax.ShapeDtypeStruct((NP, NCLASS), jnp.float32),
    )(acc2, g2, dinv2d, b2r, Wfcr, bfc2d)


def kernel(x, edge_index, W1, b1, W2, b2, Wfc, bfc):
    x = x.astype(jnp.float32)
    src = edge_index[0].astype(jnp.int32)
    dst = edge_index[1].astype(jnp.int32)
    srcp = jnp.concatenate([src, jnp.zeros((EP - E,), jnp.int32)])
    dstp = jnp.concatenate([dst, jnp.full((EP - E,), N + 16, jnp.int32)])
    srcp = srcp.reshape(EP // CHUNK, CHUNK)
    dstp = dstp.reshape(EP // CHUNK, CHUNK)
    xp = jnp.zeros((NP, NFEAT), jnp.float32).at[:N].set(x)

    cnta, cntb = _deg_cnt(dstp)

    g1, dinv2d = _mm1(xp, W1, cnta[:, None], cntb[:, None])
    acc1 = _msg_pass(g1.reshape(NC * NP, 128), srcp, dstp)
    g2 = _mm2(acc1.reshape(2, NP, 128), g1, dinv2d,
              b1.reshape(2, 1, 128), W2.reshape(2, 128, NHID))
    acc2 = _msg_pass(g2.reshape(NC * NP, 128), srcp, dstp)
    out = _mm3(acc2.reshape(2, NP, 128), g2, dinv2d,
               b2.reshape(2, 1, 128), Wfc.reshape(2, 128, NCLASS),
               bfc.reshape(1, NCLASS))
    return out[:N]


# speedup vs baseline: 1.2008x; 1.0015x over previous
import functools

import jax
import jax.numpy as jnp
from jax import lax
from jax.experimental import pallas as pl
from jax.experimental.pallas import tpu as pltpu
from jax.experimental.pallas import tpu_sc as plsc

N = 10000
NP = 10240
E = 160000
EP = 163840
NFEAT = 256
NHID = 256
NCLASS = 40
NC, NS, L = 2, 16, 16
CHUNK = 128
EDGES_PER_TILE = EP // NS
NCHUNK = EDGES_PER_TILE // CHUNK
ROWS_PER_TILE = NP // NS

_mesh = plsc.VectorSubcoreMesh(core_axis_name="c", subcore_axis_name="s")


NCHUNK_HALF = NCHUNK // 2


@functools.partial(
    pl.kernel,
    mesh=_mesh,
    out_type=[jax.ShapeDtypeStruct((NP,), jnp.float32),
              jax.ShapeDtypeStruct((NP,), jnp.float32)],
    scratch_types=[
        pltpu.MemorySpace.VMEM_SHARED((NP, 128), jnp.float32),
        pltpu.VMEM((CHUNK, 128), jnp.float32),
        pltpu.VMEM((CHUNK, 128), jnp.float32),
        pltpu.VMEM((NCHUNK_HALF, CHUNK), jnp.int32),
        pltpu.VMEM((2 * L, 128), jnp.float32),
        pltpu.VMEM((ROWS_PER_TILE,), jnp.float32),
        pltpu.SemaphoreType.DMA,
        pltpu.SemaphoreType.DMA,
        pltpu.SemaphoreType.DMA,
    ],
)
def _deg_cnt(dst_hbm, cnta_hbm, cntb_hbm, cnt_sh, ones_v, zeros_v, didx_v,
             slab_v, dvec_v, sem, zsem, isem):
    c = lax.axis_index("c")
    s = lax.axis_index("s")

    row_base = pl.multiple_of((c * NS + s) * NCHUNK_HALF, 8)
    pltpu.async_copy(dst_hbm.at[pl.ds(row_base, NCHUNK_HALF)], didx_v, isem)

    @pl.loop(0, CHUNK)
    def _(i):
        for j in range(128 // L):
            zeros_v[i, pl.ds(j * L, L)] = jnp.zeros((L,), jnp.float32)

    r0 = s * ROWS_PER_TILE

    @pl.loop(0, ROWS_PER_TILE // CHUNK)
    def _(i):
        pltpu.async_copy(zeros_v, cnt_sh.at[pl.ds(r0 + i * CHUNK, CHUNK)], zsem)

    @pl.loop(0, CHUNK)
    def _(i):
        for j in range(128 // L):
            ones_v[i, pl.ds(j * L, L)] = jnp.ones((L,), jnp.float32)

    @pl.loop(0, ROWS_PER_TILE // CHUNK)
    def _(i):
        pltpu.make_async_copy(zeros_v, cnt_sh.at[pl.ds(r0 + i * CHUNK, CHUNK)],
                              zsem).wait()

    pltpu.make_async_copy(dst_hbm.at[pl.ds(row_base, NCHUNK_HALF)], didx_v,
                          isem).wait()

    plsc.subcore_barrier()

    @pl.loop(0, NCHUNK_HALF)
    def _(k):
        pltpu.async_copy(ones_v, cnt_sh.at[didx_v.at[k]], sem, add=True)

    @pl.loop(0, NCHUNK_HALF)
    def _(k):
        pltpu.make_async_copy(ones_v, cnt_sh.at[didx_v.at[k]], sem).wait()

    plsc.subcore_barrier()

    out0 = s * ROWS_PER_TILE
    lane = lax.iota(jnp.int32, L)

    @pl.loop(0, ROWS_PER_TILE // (2 * L))
    def _(t):
        pltpu.sync_copy(cnt_sh.at[pl.ds(out0 + t * (2 * L), 2 * L)], slab_v)
        for half in range(2):
            d = jnp.zeros((L,), jnp.float32)
            for l in range(L):
                d = jnp.where(lane == l, slab_v[half * L + l, pl.ds(0, L)], d)
            dvec_v[pl.ds(t * (2 * L) + half * L, L)] = d

    @pl.when(c == 0)
    def _():
        pltpu.sync_copy(dvec_v, cnta_hbm.at[pl.ds(out0, ROWS_PER_TILE)])

    @pl.when(c == 1)
    def _():
        pltpu.sync_copy(dvec_v, cntb_hbm.at[pl.ds(out0, ROWS_PER_TILE)])


CPS = 16


@functools.partial(
    pl.kernel,
    mesh=_mesh,
    out_type=jax.ShapeDtypeStruct((NC * NP, 128), jnp.float32),
    scratch_types=[
        pltpu.MemorySpace.VMEM_SHARED((NP, 128), jnp.float32),
        pltpu.VMEM((CHUNK, 128), jnp.float32),
        pltpu.VMEM((CHUNK, 128), jnp.float32),
        pltpu.VMEM((2 * CPS, CHUNK), jnp.int32),
        pltpu.VMEM((2 * CPS, CHUNK), jnp.int32),
        pltpu.SemaphoreType.DMA,
        pltpu.SemaphoreType.DMA,
        pltpu.SemaphoreType.DMA,
        pltpu.SemaphoreType.DMA,
        pltpu.SemaphoreType.DMA,
    ],
)
def _msg_pass(h_hbm, src_hbm, dst_hbm, out_hbm, acc_sh, rows_a, rows_b,
              sidx_v, didx_v, gsa, gsb, ssa, ssb, ists):
    c = lax.axis_index("c")
    s = lax.axis_index("s")

    @pl.loop(0, CHUNK)
    def _(i):
        for j in range(128 // L):
            rows_a[i, pl.ds(j * L, L)] = jnp.zeros((L,), jnp.float32)

    r0 = s * ROWS_PER_TILE

    @pl.loop(0, ROWS_PER_TILE // CHUNK)
    def _(i):
        pltpu.sync_copy(rows_a, acc_sh.at[pl.ds(r0 + i * CHUNK, CHUNK)])

    plsc.subcore_barrier()

    row_base = s * (EDGES_PER_TILE // CHUNK)
    hoff = c * NP

    def stage_refs(sup):
        slot = pl.multiple_of((sup % 2) * CPS, CPS)
        off = pl.multiple_of(row_base + sup * CPS, CPS)
        return (src_hbm.at[pl.ds(off, CPS)], sidx_v.at[pl.ds(slot, CPS)],
                dst_hbm.at[pl.ds(off, CPS)], didx_v.at[pl.ds(slot, CPS)])

    def stage_start(sup):
        s_src, d_src, s_dst, d_dst = stage_refs(sup)
        pltpu.async_copy(s_src, d_src, ists)
        pltpu.async_copy(s_dst, d_dst, ists)

    def stage_finish(sup):
        s_src, d_src, s_dst, d_dst = stage_refs(sup)
        pltpu.make_async_copy(s_src, d_src, ists).wait()
        pltpu.make_async_copy(s_dst, d_dst, ists).wait()
        slot = (sup % 2) * CPS

        @pl.loop(0, CPS)
        def _(j):
            for jj in range(CHUNK // L):
                sidx_v[slot + j, pl.ds(jj * L, L)] = (
                    sidx_v[slot + j, pl.ds(jj * L, L)] + hoff)

    def idx_row(k):
        return ((k // CPS) % 2) * CPS + (k % CPS)

    def gather(k, buf, sem):
        pltpu.async_copy(h_hbm.at[sidx_v.at[idx_row(k)]], buf, sem)

    def scatter(k, buf, sem):
        pltpu.async_copy(buf, acc_sh.at[didx_v.at[idx_row(k)]], sem, add=True)

    stage_start(0)
    stage_finish(0)
    gather(0, rows_a, gsa)

    @pl.loop(0, NCHUNK // 2)
    def _(t):
        k = t * 2
        pltpu.make_async_copy(h_hbm.at[sidx_v.at[idx_row(k)]], rows_a, gsa).wait()

        @pl.when(t > 0)
        def _():
            pltpu.make_async_copy(rows_b, acc_sh.at[didx_v.at[idx_row(k)]],
                                  ssb).wait()

        @pl.when((k % CPS == 0) & (k < NCHUNK - CPS))
        def _():
            stage_start(k // CPS + 1)

        gather(k + 1, rows_b, gsb)
        scatter(k, rows_a, ssa)

        pltpu.make_async_copy(h_hbm.at[sidx_v.at[idx_row(k + 1)]], rows_b,
                              gsb).wait()
        scatter(k + 1, rows_b, ssb)

        pltpu.make_async_copy(rows_a, acc_sh.at[didx_v.at[idx_row(k)]],
                              ssa).wait()

        @pl.when((k % CPS == CPS - 2) & (k + 2 < NCHUNK))
        def _():
            stage_finish((k + 2) // CPS)

        @pl.when(k + 2 < NCHUNK)
        def _():
            gather(k + 2, rows_a, gsa)

    pltpu.make_async_copy(rows_b, acc_sh.at[didx_v.at[idx_row(NCHUNK - 1)]],
                          ssb).wait()

    plsc.subcore_barrier()
    pltpu.sync_copy(acc_sh.at[pl.ds(r0, ROWS_PER_TILE)],
                    out_hbm.at[pl.ds(c * NP + r0, ROWS_PER_TILE)])


BR = 2048


def _mm1_body(x_ref, w_ref, ca_ref, cb_ref, out_ref, dinv_ref):
    h = jnp.dot(x_ref[...], w_ref[...], preferred_element_type=jnp.float32)
    d = lax.rsqrt(ca_ref[...] + cb_ref[...] + 1.0)
    dinv_ref[...] = d
    out_ref[0] = h[:, :128] * d
    out_ref[1] = h[:, 128:] * d


def _mm1(xp, W1, cA, cB):
    return pl.pallas_call(
        _mm1_body,
        grid=(NP // BR,),
        in_specs=[
            pl.BlockSpec((BR, NFEAT), lambda i: (i, 0)),
            pl.BlockSpec((NFEAT, NHID), lambda i: (0, 0)),
            pl.BlockSpec((BR, 1), lambda i: (i, 0)),
            pl.BlockSpec((BR, 1), lambda i: (i, 0)),
        ],
        out_specs=[
            pl.BlockSpec((2, BR, 128), lambda i: (0, i, 0)),
            pl.BlockSpec((BR, 1), lambda i: (i, 0)),
        ],
        out_shape=[
            jax.ShapeDtypeStruct((2, NP, 128), jnp.float32),
            jax.ShapeDtypeStruct((NP, 1), jnp.float32),
        ],
    )(xp, W1, cA, cB)


def _mm2_body(acc_ref, g_ref, dinv_ref, b_ref, w_ref, out_ref):
    dinv = dinv_ref[...]
    a0 = jnp.maximum((acc_ref[0] + g_ref[0]) * dinv + b_ref[0, 0], 0.0)
    a1 = jnp.maximum((acc_ref[1] + g_ref[1]) * dinv + b_ref[1, 0], 0.0)
    p = (jnp.dot(a0, w_ref[0], preferred_element_type=jnp.float32)
         + jnp.dot(a1, w_ref[1], preferred_element_type=jnp.float32))
    out_ref[0] = p * dinv


def _mm2(acc1, g1, dinv2d, b1r, W2r):
    return pl.pallas_call(
        _mm2_body,
        grid=(NP // BR, 2),
        in_specs=[
            pl.BlockSpec((2, BR, 128), lambda i, j: (0, i, 0)),
            pl.BlockSpec((2, BR, 128), lambda i, j: (0, i, 0)),
            pl.BlockSpec((BR, 1), lambda i, j: (i, 0)),
            pl.BlockSpec((2, 1, 128), lambda i, j: (0, 0, 0)),
            pl.BlockSpec((2, 128, 128), lambda i, j: (0, 0, j)),
        ],
        out_specs=pl.BlockSpec((1, BR, 128), lambda i, j: (j, i, 0)),
        out_shape=jax.ShapeDtypeStruct((2, NP, 128), jnp.float32),
    )(acc1, g1, dinv2d, b1r, W2r)


def _mm3_body(acc_ref, g_ref, dinv_ref, b_ref, w_ref, bfc_ref, out_ref):
    dinv = dinv_ref[...]
    a0 = (acc_ref[0] + g_ref[0]) * dinv + b_ref[0, 0]
    a1 = (acc_ref[1] + g_ref[1]) * dinv + b_ref[1, 0]
    out_ref[...] = (jnp.dot(a0, w_ref[0], preferred_element_type=jnp.float32)
                    + jnp.dot(a1, w_ref[1], preferred_element_type=jnp.float32)
                    + jnp.broadcast_to(bfc_ref[...], out_ref.shape))


def _mm3(acc2, g2, dinv2d, b2r, Wfcr, bfc2d):
    return pl.pallas_call(
        _mm3_body,
        grid=(NP // BR,),
        in_specs=[
            pl.BlockSpec((2, BR, 128), lambda i: (0, i, 0)),
            pl.BlockSpec((2, BR, 128), lambda i: (0, i, 0)),
            pl.BlockSpec((BR, 1), lambda i: (i, 0)),
            pl.BlockSpec((2, 1, 128), lambda i: (0, 0, 0)),
            pl.BlockSpec((2, 128, NCLASS), lambda i: (0, 0, 0)),
            pl.BlockSpec((1, NCLASS), lambda i: (0, 0)),
        ],
        out_specs=pl.BlockSpec((BR, NCLASS), lambda i: (i, 0)),
        out_shape=jax.ShapeDtypeStruct((NP, NCLASS), jnp.float32),
    )(acc2, g2, dinv2d, b2r, Wfcr, bfc2d)


def kernel(x, edge_index, W1, b1, W2, b2, Wfc, bfc):
    x = x.astype(jnp.float32)
    src = edge_index[0].astype(jnp.int32)
    dst = edge_index[1].astype(jnp.int32)
    srcp = jnp.concatenate([src, jnp.zeros((EP - E,), jnp.int32)])
    dstp = jnp.concatenate([dst, jnp.full((EP - E,), N + 16, jnp.int32)])
    srcp = srcp.reshape(EP // CHUNK, CHUNK)
    dstp = dstp.reshape(EP // CHUNK, CHUNK)
    xp = jnp.zeros((NP, NFEAT), jnp.float32).at[:N].set(x)

    cnta, cntb = _deg_cnt(dstp)

    g1, dinv2d = _mm1(xp, W1, cnta[:, None], cntb[:, None])
    acc1 = _msg_pass(g1.reshape(NC * NP, 128), srcp, dstp)
    g2 = _mm2(acc1.reshape(2, NP, 128), g1, dinv2d,
              b1.reshape(2, 1, 128), W2.reshape(2, 128, NHID))
    acc2 = _msg_pass(g2.reshape(NC * NP, 128), srcp, dstp)
    out = _mm3(acc2.reshape(2, NP, 128), g2, dinv2d,
               b2.reshape(2, 1, 128), Wfc.reshape(2, 128, NCLASS),
               bfc.reshape(1, NCLASS))
    return out[:N]


# speedup vs baseline: 1.2013x; 1.0005x over previous
import functools

import jax
import jax.numpy as jnp
from jax import lax
from jax.experimental import pallas as pl
from jax.experimental.pallas import tpu as pltpu
from jax.experimental.pallas import tpu_sc as plsc

N = 10000
NP = 10240
E = 160000
EP = 163840
NFEAT = 256
NHID = 256
NCLASS = 40
NC, NS, L = 2, 16, 16
CHUNK = 128
EDGES_PER_TILE = EP // NS
NCHUNK = EDGES_PER_TILE // CHUNK
ROWS_PER_TILE = NP // NS

_mesh = plsc.VectorSubcoreMesh(core_axis_name="c", subcore_axis_name="s")


NCHUNK_HALF = NCHUNK // 2


@functools.partial(
    pl.kernel,
    mesh=_mesh,
    out_type=[jax.ShapeDtypeStruct((NP,), jnp.float32),
              jax.ShapeDtypeStruct((NP,), jnp.float32)],
    scratch_types=[
        pltpu.MemorySpace.VMEM_SHARED((NP, 128), jnp.float32),
        pltpu.VMEM((CHUNK, 128), jnp.float32),
        pltpu.VMEM((CHUNK, 128), jnp.float32),
        pltpu.VMEM((NCHUNK_HALF, CHUNK), jnp.int32),
        pltpu.VMEM((2 * L, 128), jnp.float32),
        pltpu.VMEM((ROWS_PER_TILE,), jnp.float32),
        pltpu.SemaphoreType.DMA,
        pltpu.SemaphoreType.DMA,
        pltpu.SemaphoreType.DMA,
    ],
)
def _deg_cnt(dst_hbm, cnta_hbm, cntb_hbm, cnt_sh, ones_v, zeros_v, didx_v,
             slab_v, dvec_v, sem, zsem, isem):
    c = lax.axis_index("c")
    s = lax.axis_index("s")

    row_base = pl.multiple_of((c * NS + s) * NCHUNK_HALF, 8)
    pltpu.async_copy(dst_hbm.at[pl.ds(row_base, NCHUNK_HALF)], didx_v, isem)

    @pl.loop(0, CHUNK)
    def _(i):
        for j in range(128 // L):
            zeros_v[i, pl.ds(j * L, L)] = jnp.zeros((L,), jnp.float32)

    r0 = s * ROWS_PER_TILE

    @pl.loop(0, ROWS_PER_TILE // CHUNK)
    def _(i):
        pltpu.async_copy(zeros_v, cnt_sh.at[pl.ds(r0 + i * CHUNK, CHUNK)], zsem)

    @pl.loop(0, CHUNK)
    def _(i):
        for j in range(128 // L):
            ones_v[i, pl.ds(j * L, L)] = jnp.ones((L,), jnp.float32)

    @pl.loop(0, ROWS_PER_TILE // CHUNK)
    def _(i):
        pltpu.make_async_copy(zeros_v, cnt_sh.at[pl.ds(r0 + i * CHUNK, CHUNK)],
                              zsem).wait()

    pltpu.make_async_copy(dst_hbm.at[pl.ds(row_base, NCHUNK_HALF)], didx_v,
                          isem).wait()

    plsc.subcore_barrier()

    @pl.loop(0, NCHUNK_HALF)
    def _(k):
        pltpu.async_copy(ones_v, cnt_sh.at[didx_v.at[k]], sem, add=True)

    @pl.loop(0, NCHUNK_HALF)
    def _(k):
        pltpu.make_async_copy(ones_v, cnt_sh.at[didx_v.at[k]], sem).wait()

    plsc.subcore_barrier()

    out0 = s * ROWS_PER_TILE
    lane = lax.iota(jnp.int32, L)

    @pl.loop(0, ROWS_PER_TILE // (2 * L))
    def _(t):
        pltpu.sync_copy(cnt_sh.at[pl.ds(out0 + t * (2 * L), 2 * L)], slab_v)
        for half in range(2):
            d = jnp.zeros((L,), jnp.float32)
            for l in range(L):
                d = jnp.where(lane == l, slab_v[half * L + l, pl.ds(0, L)], d)
            dvec_v[pl.ds(t * (2 * L) + half * L, L)] = d

    @pl.when(c == 0)
    def _():
        pltpu.sync_copy(dvec_v, cnta_hbm.at[pl.ds(out0, ROWS_PER_TILE)])

    @pl.when(c == 1)
    def _():
        pltpu.sync_copy(dvec_v, cntb_hbm.at[pl.ds(out0, ROWS_PER_TILE)])


CPS = 16


@functools.partial(
    pl.kernel,
    mesh=_mesh,
    out_type=jax.ShapeDtypeStruct((NC * NP, 128), jnp.float32),
    scratch_types=[
        pltpu.MemorySpace.VMEM_SHARED((NP, 128), jnp.float32),
        pltpu.VMEM((CHUNK, 128), jnp.float32),
        pltpu.VMEM((CHUNK, 128), jnp.float32),
        pltpu.VMEM((2 * CPS, CHUNK), jnp.int32),
        pltpu.VMEM((2 * CPS, CHUNK), jnp.int32),
        pltpu.SemaphoreType.DMA,
        pltpu.SemaphoreType.DMA,
        pltpu.SemaphoreType.DMA,
        pltpu.SemaphoreType.DMA,
        pltpu.SemaphoreType.DMA,
    ],
)
def _msg_pass(h_hbm, src_hbm, dst_hbm, out_hbm, acc_sh, rows_a, rows_b,
              sidx_v, didx_v, gsa, gsb, ssa, ssb, ists):
    c = lax.axis_index("c")
    s = lax.axis_index("s")

    @pl.loop(0, CHUNK)
    def _(i):
        for j in range(128 // L):
            rows_a[i, pl.ds(j * L, L)] = jnp.zeros((L,), jnp.float32)

    r0 = s * ROWS_PER_TILE

    @pl.loop(0, ROWS_PER_TILE // CHUNK)
    def _(i):
        pltpu.sync_copy(rows_a, acc_sh.at[pl.ds(r0 + i * CHUNK, CHUNK)])

    plsc.subcore_barrier()

    row_base = s * (EDGES_PER_TILE // CHUNK)
    hoff = c * NP

    def stage_refs(sup):
        slot = pl.multiple_of((sup % 2) * CPS, CPS)
        off = pl.multiple_of(row_base + sup * CPS, CPS)
        return (src_hbm.at[pl.ds(off, CPS)], sidx_v.at[pl.ds(slot, CPS)],
                dst_hbm.at[pl.ds(off, CPS)], didx_v.at[pl.ds(slot, CPS)])

    def stage_start(sup):
        s_src, d_src, s_dst, d_dst = stage_refs(sup)
        pltpu.async_copy(s_src, d_src, ists)
        pltpu.async_copy(s_dst, d_dst, ists)

    def stage_finish(sup):
        s_src, d_src, s_dst, d_dst = stage_refs(sup)
        pltpu.make_async_copy(s_src, d_src, ists).wait()
        pltpu.make_async_copy(s_dst, d_dst, ists).wait()
        slot = (sup % 2) * CPS

        @pl.loop(0, CPS)
        def _(j):
            for jj in range(CHUNK // L):
                sidx_v[slot + j, pl.ds(jj * L, L)] = (
                    sidx_v[slot + j, pl.ds(jj * L, L)] + hoff)

    def idx_row(k):
        return ((k // CPS) % 2) * CPS + (k % CPS)

    def gather(k, buf, sem):
        pltpu.async_copy(h_hbm.at[sidx_v.at[idx_row(k)]], buf, sem)

    def scatter(k, buf, sem):
        pltpu.async_copy(buf, acc_sh.at[didx_v.at[idx_row(k)]], sem, add=True,
                         priority=1)

    stage_start(0)
    stage_finish(0)
    gather(0, rows_a, gsa)

    @pl.loop(0, NCHUNK // 2)
    def _(t):
        k = t * 2
        pltpu.make_async_copy(h_hbm.at[sidx_v.at[idx_row(k)]], rows_a, gsa).wait()

        @pl.when(t > 0)
        def _():
            pltpu.make_async_copy(rows_b, acc_sh.at[didx_v.at[idx_row(k)]],
                                  ssb).wait()

        @pl.when((k % CPS == 0) & (k < NCHUNK - CPS))
        def _():
            stage_start(k // CPS + 1)

        gather(k + 1, rows_b, gsb)
        scatter(k, rows_a, ssa)

        pltpu.make_async_copy(h_hbm.at[sidx_v.at[idx_row(k + 1)]], rows_b,
                              gsb).wait()
        scatter(k + 1, rows_b, ssb)

        pltpu.make_async_copy(rows_a, acc_sh.at[didx_v.at[idx_row(k)]],
                              ssa).wait()

        @pl.when((k % CPS == CPS - 2) & (k + 2 < NCHUNK))
        def _():
            stage_finish((k + 2) // CPS)

        @pl.when(k + 2 < NCHUNK)
        def _():
            gather(k + 2, rows_a, gsa)

    pltpu.make_async_copy(rows_b, acc_sh.at[didx_v.at[idx_row(NCHUNK - 1)]],
                          ssb).wait()

    plsc.subcore_barrier()
    pltpu.sync_copy(acc_sh.at[pl.ds(r0, ROWS_PER_TILE)],
                    out_hbm.at[pl.ds(c * NP + r0, ROWS_PER_TILE)])


BR = 2048


def _mm1_body(x_ref, w_ref, ca_ref, cb_ref, out_ref, dinv_ref):
    h = jnp.dot(x_ref[...], w_ref[...], preferred_element_type=jnp.float32)
    d = lax.rsqrt(ca_ref[...] + cb_ref[...] + 1.0)
    dinv_ref[...] = d
    out_ref[0] = h[:, :128] * d
    out_ref[1] = h[:, 128:] * d


def _mm1(xp, W1, cA, cB):
    return pl.pallas_call(
        _mm1_body,
        grid=(NP // BR,),
        in_specs=[
            pl.BlockSpec((BR, NFEAT), lambda i: (i, 0)),
            pl.BlockSpec((NFEAT, NHID), lambda i: (0, 0)),
            pl.BlockSpec((BR, 1), lambda i: (i, 0)),
            pl.BlockSpec((BR, 1), lambda i: (i, 0)),
        ],
        out_specs=[
            pl.BlockSpec((2, BR, 128), lambda i: (0, i, 0)),
            pl.BlockSpec((BR, 1), lambda i: (i, 0)),
        ],
        out_shape=[
            jax.ShapeDtypeStruct((2, NP, 128), jnp.float32),
            jax.ShapeDtypeStruct((NP, 1), jnp.float32),
        ],
    )(xp, W1, cA, cB)


def _mm2_body(acc_ref, g_ref, dinv_ref, b_ref, w_ref, out_ref):
    dinv = dinv_ref[...]
    a0 = jnp.maximum((acc_ref[0] + g_ref[0]) * dinv + b_ref[0, 0], 0.0)
    a1 = jnp.maximum((acc_ref[1] + g_ref[1]) * dinv + b_ref[1, 0], 0.0)
    p = (jnp.dot(a0, w_ref[0], preferred_element_type=jnp.float32)
         + jnp.dot(a1, w_ref[1], preferred_element_type=jnp.float32))
    out_ref[0] = p * dinv


def _mm2(acc1, g1, dinv2d, b1r, W2r):
    return pl.pallas_call(
        _mm2_body,
        grid=(NP // BR, 2),
        in_specs=[
            pl.BlockSpec((2, BR, 128), lambda i, j: (0, i, 0)),
            pl.BlockSpec((2, BR, 128), lambda i, j: (0, i, 0)),
            pl.BlockSpec((BR, 1), lambda i, j: (i, 0)),
            pl.BlockSpec((2, 1, 128), lambda i, j: (0, 0, 0)),
            pl.BlockSpec((2, 128, 128), lambda i, j: (0, 0, j)),
        ],
        out_specs=pl.BlockSpec((1, BR, 128), lambda i, j: (j, i, 0)),
        out_shape=jax.ShapeDtypeStruct((2, NP, 128), jnp.float32),
    )(acc1, g1, dinv2d, b1r, W2r)


def _mm3_body(acc_ref, g_ref, dinv_ref, b_ref, w_ref, bfc_ref, out_ref):
    dinv = dinv_ref[...]
    a0 = (acc_ref[0] + g_ref[0]) * dinv + b_ref[0, 0]
    a1 = (acc_ref[1] + g_ref[1]) * dinv + b_ref[1, 0]
    out_ref[...] = (jnp.dot(a0, w_ref[0], preferred_element_type=jnp.float32)
                    + jnp.dot(a1, w_ref[1], preferred_element_type=jnp.float32)
                    + jnp.broadcast_to(bfc_ref[...], out_ref.shape))


def _mm3(acc2, g2, dinv2d, b2r, Wfcr, bfc2d):
    return pl.pallas_call(
        _mm3_body,
        grid=(NP // BR,),
        in_specs=[
            pl.BlockSpec((2, BR, 128), lambda i: (0, i, 0)),
            pl.BlockSpec((2, BR, 128), lambda i: (0, i, 0)),
            pl.BlockSpec((BR, 1), lambda i: (i, 0)),
            pl.BlockSpec((2, 1, 128), lambda i: (0, 0, 0)),
            pl.BlockSpec((2, 128, NCLASS), lambda i: (0, 0, 0)),
            pl.BlockSpec((1, NCLASS), lambda i: (0, 0)),
        ],
        out_specs=pl.BlockSpec((BR, NCLASS), lambda i: (i, 0)),
        out_shape=jax.ShapeDtypeStruct((NP, NCLASS), jnp.float32),
    )(acc2, g2, dinv2d, b2r, Wfcr, bfc2d)


def kernel(x, edge_index, W1, b1, W2, b2, Wfc, bfc):
    x = x.astype(jnp.float32)
    src = edge_index[0].astype(jnp.int32)
    dst = edge_index[1].astype(jnp.int32)
    srcp = jnp.concatenate([src, jnp.zeros((EP - E,), jnp.int32)])
    dstp = jnp.concatenate([dst, jnp.full((EP - E,), N + 16, jnp.int32)])
    srcp = srcp.reshape(EP // CHUNK, CHUNK)
    dstp = dstp.reshape(EP // CHUNK, CHUNK)
    xp = jnp.zeros((NP, NFEAT), jnp.float32).at[:N].set(x)

    cnta, cntb = _deg_cnt(dstp)

    g1, dinv2d = _mm1(xp, W1, cnta[:, None], cntb[:, None])
    acc1 = _msg_pass(g1.reshape(NC * NP, 128), srcp, dstp)
    g2 = _mm2(acc1.reshape(2, NP, 128), g1, dinv2d,
              b1.reshape(2, 1, 128), W2.reshape(2, 128, NHID))
    acc2 = _msg_pass(g2.reshape(NC * NP, 128), srcp, dstp)
    out = _mm3(acc2.reshape(2, NP, 128), g2, dinv2d,
               b2.reshape(2, 1, 128), Wfc.reshape(2, 128, NCLASS),
               bfc.reshape(1, NCLASS))
    return out[:N]


# speedup vs baseline: 1.2018x; 1.0004x over previous
import functools

import jax
import jax.numpy as jnp
from jax import lax
from jax.experimental import pallas as pl
from jax.experimental.pallas import tpu as pltpu
from jax.experimental.pallas import tpu_sc as plsc

N = 10000
NP = 10240
E = 160000
EP = 163840
NFEAT = 256
NHID = 256
NCLASS = 40
NC, NS, L = 2, 16, 16
CHUNK = 128
EDGES_PER_TILE = EP // NS
NCHUNK = EDGES_PER_TILE // CHUNK
ROWS_PER_TILE = NP // NS

_mesh = plsc.VectorSubcoreMesh(core_axis_name="c", subcore_axis_name="s")


NCHUNK_HALF = NCHUNK // 2


@functools.partial(
    pl.kernel,
    mesh=_mesh,
    out_type=[jax.ShapeDtypeStruct((NP,), jnp.float32),
              jax.ShapeDtypeStruct((NP,), jnp.float32)],
    scratch_types=[
        pltpu.MemorySpace.VMEM_SHARED((NP, 128), jnp.float32),
        pltpu.VMEM((CHUNK, 128), jnp.float32),
        pltpu.VMEM((CHUNK, 128), jnp.float32),
        pltpu.VMEM((NCHUNK_HALF, CHUNK), jnp.int32),
        pltpu.VMEM((2 * L, 128), jnp.float32),
        pltpu.VMEM((ROWS_PER_TILE,), jnp.float32),
        pltpu.SemaphoreType.DMA,
        pltpu.SemaphoreType.DMA,
        pltpu.SemaphoreType.DMA,
    ],
)
def _deg_cnt(dst_hbm, cnta_hbm, cntb_hbm, cnt_sh, ones_v, zeros_v, didx_v,
             slab_v, dvec_v, sem, zsem, isem):
    c = lax.axis_index("c")
    s = lax.axis_index("s")

    row_base = pl.multiple_of((c * NS + s) * NCHUNK_HALF, 8)
    pltpu.async_copy(dst_hbm.at[pl.ds(row_base, NCHUNK_HALF)], didx_v, isem)

    @pl.loop(0, CHUNK)
    def _(i):
        for j in range(128 // L):
            zeros_v[i, pl.ds(j * L, L)] = jnp.zeros((L,), jnp.float32)

    r0 = s * ROWS_PER_TILE

    @pl.loop(0, ROWS_PER_TILE // CHUNK)
    def _(i):
        pltpu.async_copy(zeros_v, cnt_sh.at[pl.ds(r0 + i * CHUNK, CHUNK)], zsem)

    @pl.loop(0, CHUNK)
    def _(i):
        for j in range(128 // L):
            ones_v[i, pl.ds(j * L, L)] = jnp.ones((L,), jnp.float32)

    @pl.loop(0, ROWS_PER_TILE // CHUNK)
    def _(i):
        pltpu.make_async_copy(zeros_v, cnt_sh.at[pl.ds(r0 + i * CHUNK, CHUNK)],
                              zsem).wait()

    pltpu.make_async_copy(dst_hbm.at[pl.ds(row_base, NCHUNK_HALF)], didx_v,
                          isem).wait()

    plsc.subcore_barrier()

    @pl.loop(0, NCHUNK_HALF)
    def _(k):
        pltpu.async_copy(ones_v, cnt_sh.at[didx_v.at[k]], sem, add=True)

    @pl.loop(0, NCHUNK_HALF)
    def _(k):
        pltpu.make_async_copy(ones_v, cnt_sh.at[didx_v.at[k]], sem).wait()

    plsc.subcore_barrier()

    out0 = s * ROWS_PER_TILE
    lane = lax.iota(jnp.int32, L)

    @pl.loop(0, ROWS_PER_TILE // (2 * L))
    def _(t):
        pltpu.sync_copy(cnt_sh.at[pl.ds(out0 + t * (2 * L), 2 * L)], slab_v)
        for half in range(2):
            d = jnp.zeros((L,), jnp.float32)
            for l in range(L):
                d = jnp.where(lane == l, slab_v[half * L + l, pl.ds(0, L)], d)
            dvec_v[pl.ds(t * (2 * L) + half * L, L)] = d

    @pl.when(c == 0)
    def _():
        pltpu.sync_copy(dvec_v, cnta_hbm.at[pl.ds(out0, ROWS_PER_TILE)])

    @pl.when(c == 1)
    def _():
        pltpu.sync_copy(dvec_v, cntb_hbm.at[pl.ds(out0, ROWS_PER_TILE)])


CPS = 16


@functools.partial(
    pl.kernel,
    mesh=_mesh,
    out_type=jax.ShapeDtypeStruct((NC * NP, 128), jnp.float32),
    scratch_types=[
        pltpu.MemorySpace.VMEM_SHARED((NP, 128), jnp.float32),
        pltpu.VMEM((CHUNK, 128), jnp.float32),
        pltpu.VMEM((CHUNK, 128), jnp.float32),
        pltpu.VMEM((2 * CPS, CHUNK), jnp.int32),
        pltpu.VMEM((2 * CPS, CHUNK), jnp.int32),
        pltpu.SemaphoreType.DMA,
        pltpu.SemaphoreType.DMA,
        pltpu.SemaphoreType.DMA,
        pltpu.SemaphoreType.DMA,
        pltpu.SemaphoreType.DMA,
    ],
)
def _msg_pass(h_hbm, src_hbm, dst_hbm, out_hbm, acc_sh, rows_a, rows_b,
              sidx_v, didx_v, gsa, gsb, ssa, ssb, ists):
    c = lax.axis_index("c")
    s = lax.axis_index("s")

    @pl.loop(0, CHUNK)
    def _(i):
        for j in range(128 // L):
            rows_a[i, pl.ds(j * L, L)] = jnp.zeros((L,), jnp.float32)

    r0 = s * ROWS_PER_TILE

    @pl.loop(0, ROWS_PER_TILE // CHUNK)
    def _(i):
        pltpu.sync_copy(rows_a, acc_sh.at[pl.ds(r0 + i * CHUNK, CHUNK)])

    plsc.subcore_barrier()

    row_base = s * (EDGES_PER_TILE // CHUNK)
    hoff = c * NP

    def stage_refs(sup):
        slot = pl.multiple_of((sup % 2) * CPS, CPS)
        off = pl.multiple_of(row_base + sup * CPS, CPS)
        return (src_hbm.at[pl.ds(off, CPS)], sidx_v.at[pl.ds(slot, CPS)],
                dst_hbm.at[pl.ds(off, CPS)], didx_v.at[pl.ds(slot, CPS)])

    def stage_start(sup):
        s_src, d_src, s_dst, d_dst = stage_refs(sup)
        pltpu.async_copy(s_src, d_src, ists)
        pltpu.async_copy(s_dst, d_dst, ists)

    def stage_finish(sup):
        s_src, d_src, s_dst, d_dst = stage_refs(sup)
        pltpu.make_async_copy(s_src, d_src, ists).wait()
        pltpu.make_async_copy(s_dst, d_dst, ists).wait()
        slot = (sup % 2) * CPS

        @pl.loop(0, CPS)
        def _(j):
            for jj in range(CHUNK // L):
                sidx_v[slot + j, pl.ds(jj * L, L)] = (
                    sidx_v[slot + j, pl.ds(jj * L, L)] + hoff)

    def idx_row(k):
        return ((k // CPS) % 2) * CPS + (k % CPS)

    def gather(k, buf, sem):
        pltpu.async_copy(h_hbm.at[sidx_v.at[idx_row(k)]], buf, sem)

    def scatter(k, buf, sem):
        pltpu.async_copy(buf, acc_sh.at[didx_v.at[idx_row(k)]], sem, add=True)

    stage_start(0)
    stage_finish(0)
    gather(0, rows_a, gsa)

    @pl.loop(0, NCHUNK // 2)
    def _(t):
        k = t * 2
        pltpu.make_async_copy(h_hbm.at[sidx_v.at[idx_row(k)]], rows_a, gsa).wait()

        @pl.when(t > 0)
        def _():
            pltpu.make_async_copy(rows_b, acc_sh.at[didx_v.at[idx_row(k)]],
                                  ssb).wait()

        @pl.when((k % CPS == 0) & (k < NCHUNK - CPS))
        def _():
            stage_start(k // CPS + 1)

        gather(k + 1, rows_b, gsb)
        scatter(k, rows_a, ssa)

        pltpu.make_async_copy(h_hbm.at[sidx_v.at[idx_row(k + 1)]], rows_b,
                              gsb).wait()
        scatter(k + 1, rows_b, ssb)

        pltpu.make_async_copy(rows_a, acc_sh.at[didx_v.at[idx_row(k)]],
                              ssa).wait()

        @pl.when((k % CPS == CPS - 2) & (k + 2 < NCHUNK))
        def _():
            stage_finish((k + 2) // CPS)

        @pl.when(k + 2 < NCHUNK)
        def _():
            gather(k + 2, rows_a, gsa)

    pltpu.make_async_copy(rows_b, acc_sh.at[didx_v.at[idx_row(NCHUNK - 1)]],
                          ssb).wait()

    plsc.subcore_barrier()
    pltpu.sync_copy(acc_sh.at[pl.ds(r0, ROWS_PER_TILE)],
                    out_hbm.at[pl.ds(c * NP + r0, ROWS_PER_TILE)])


BR = 2048


def _mm1_body(x_ref, w_ref, ca_ref, cb_ref, out_ref, dinv_ref):
    h = jnp.dot(x_ref[...], w_ref[...], preferred_element_type=jnp.float32)
    d = lax.rsqrt(ca_ref[...] + cb_ref[...] + 1.0)
    dinv_ref[...] = d
    out_ref[0] = h[:, :128] * d
    out_ref[1] = h[:, 128:] * d


def _mm1(xp, W1, cA, cB):
    return pl.pallas_call(
        _mm1_body,
        grid=(NP // BR,),
        in_specs=[
            pl.BlockSpec((BR, NFEAT), lambda i: (i, 0)),
            pl.BlockSpec((NFEAT, NHID), lambda i: (0, 0)),
            pl.BlockSpec((BR, 1), lambda i: (i, 0)),
            pl.BlockSpec((BR, 1), lambda i: (i, 0)),
        ],
        out_specs=[
            pl.BlockSpec((2, BR, 128), lambda i: (0, i, 0)),
            pl.BlockSpec((BR, 1), lambda i: (i, 0)),
        ],
        out_shape=[
            jax.ShapeDtypeStruct((2, NP, 128), jnp.float32),
            jax.ShapeDtypeStruct((NP, 1), jnp.float32),
        ],
    )(xp, W1, cA, cB)


def _mm2_body(acc_ref, g_ref, dinv_ref, b_ref, w_ref, out_ref):
    dinv = dinv_ref[...]
    a0 = jnp.maximum((acc_ref[0] + g_ref[0]) * dinv + b_ref[0, 0], 0.0)
    a1 = jnp.maximum((acc_ref[1] + g_ref[1]) * dinv + b_ref[1, 0], 0.0)
    p = (jnp.dot(a0, w_ref[0], preferred_element_type=jnp.float32)
         + jnp.dot(a1, w_ref[1], preferred_element_type=jnp.float32))
    out_ref[0] = p * dinv


def _mm2(acc1, g1, dinv2d, b1r, W2r):
    return pl.pallas_call(
        _mm2_body,
        grid=(NP // BR, 2),
        in_specs=[
            pl.BlockSpec((2, BR, 128), lambda i, j: (0, i, 0)),
            pl.BlockSpec((2, BR, 128), lambda i, j: (0, i, 0)),
            pl.BlockSpec((BR, 1), lambda i, j: (i, 0)),
            pl.BlockSpec((2, 1, 128), lambda i, j: (0, 0, 0)),
            pl.BlockSpec((2, 128, 128), lambda i, j: (0, 0, j)),
        ],
        out_specs=pl.BlockSpec((1, BR, 128), lambda i, j: (j, i, 0)),
        out_shape=jax.ShapeDtypeStruct((2, NP, 128), jnp.float32),
    )(acc1, g1, dinv2d, b1r, W2r)


def _mm3_body(acc_ref, g_ref, dinv_ref, b_ref, w_ref, bfc_ref, out_ref):
    dinv = dinv_ref[...]
    a0 = (acc_ref[0] + g_ref[0]) * dinv + b_ref[0, 0]
    a1 = (acc_ref[1] + g_ref[1]) * dinv + b_ref[1, 0]
    out_ref[...] = (jnp.dot(a0, w_ref[0], preferred_element_type=jnp.float32)
                    + jnp.dot(a1, w_ref[1], preferred_element_type=jnp.float32)
                    + jnp.broadcast_to(bfc_ref[...], out_ref.shape))


def _mm3(acc2, g2, dinv2d, b2r, Wfcr, bfc2d):
    return pl.pallas_call(
        _mm3_body,
        grid=(NP // BR,),
        in_specs=[
            pl.BlockSpec((2, BR, 128), lambda i: (0, i, 0)),
            pl.BlockSpec((2, BR, 128), lambda i: (0, i, 0)),
            pl.BlockSpec((BR, 1), lambda i: (i, 0)),
            pl.BlockSpec((2, 1, 128), lambda i: (0, 0, 0)),
            pl.BlockSpec((2, 128, NCLASS), lambda i: (0, 0, 0)),
            pl.BlockSpec((1, NCLASS), lambda i: (0, 0)),
        ],
        out_specs=pl.BlockSpec((BR, NCLASS), lambda i: (i, 0)),
        out_shape=jax.ShapeDtypeStruct((NP, NCLASS), jnp.float32),
    )(acc2, g2, dinv2d, b2r, Wfcr, bfc2d)


def kernel(x, edge_index, W1, b1, W2, b2, Wfc, bfc):
    x = x.astype(jnp.float32)
    src = edge_index[0].astype(jnp.int32)
    dst = edge_index[1].astype(jnp.int32)
    srcp = jnp.concatenate([src, jnp.zeros((EP - E,), jnp.int32)])
    dstp = jnp.concatenate([dst, jnp.full((EP - E,), N + 16, jnp.int32)])
    srcp = srcp.reshape(EP // CHUNK, CHUNK)
    dstp = dstp.reshape(EP // CHUNK, CHUNK)
    xp = jnp.zeros((NP, NFEAT), jnp.float32).at[:N].set(x)

    cnta, cntb = _deg_cnt(dstp)

    g1, dinv2d = _mm1(xp, W1, cnta[:, None], cntb[:, None])
    acc1 = _msg_pass(g1.reshape(NC * NP, 128), srcp, dstp)
    g2 = _mm2(acc1.reshape(2, NP, 128), g1, dinv2d,
              b1.reshape(2, 1, 128), W2.reshape(2, 128, NHID))
    acc2 = _msg_pass(g2.reshape(NC * NP, 128), srcp, dstp)
    out = _mm3(acc2.reshape(2, NP, 128), g2, dinv2d,
               b2.reshape(2, 1, 128), Wfc.reshape(2, 128, NCLASS),
               bfc.reshape(1, NCLASS))
    return out[:N]
